# baseline (speedup 1.0000x reference)
import functools

import jax
import jax.numpy as jnp
from jax import lax
from jax.experimental import pallas as pl
from jax.experimental.pallas import tpu as pltpu
from jax.experimental.pallas import tpu_sc as plsc

N_NODES_K = 10000
N_EDGES_K = 320000
D_FEAT_K = 128
D_EDGE_K = 16
D_OUT_K = 128


def _tables_body(nodes_ref, wr_ref, ws_ref, wg_ref, g_ref, b_ref,
                 prer_ref, pres_ref):
    gvec = jnp.dot(g_ref[...], wg_ref[...],
                   preferred_element_type=jnp.float32) + b_ref[...]
    n = nodes_ref[...]
    prer_ref[...] = jnp.dot(n, wr_ref[...],
                            preferred_element_type=jnp.float32) + gvec
    pres_ref[...] = jnp.dot(n, ws_ref[...],
                            preferred_element_type=jnp.float32)


def _make_tables(nodes, w_r, w_s, w_g, g, b):
    blk = 2000
    grid = N_NODES_K // blk
    return pl.pallas_call(
        _tables_body,
        grid=(grid,),
        in_specs=[
            pl.BlockSpec((blk, D_FEAT_K), lambda i: (i, 0)),
            pl.BlockSpec((D_FEAT_K, D_OUT_K), lambda i: (0, 0)),
            pl.BlockSpec((D_FEAT_K, D_OUT_K), lambda i: (0, 0)),
            pl.BlockSpec((D_FEAT_K, D_OUT_K), lambda i: (0, 0)),
            pl.BlockSpec((1, D_FEAT_K), lambda i: (0, 0)),
            pl.BlockSpec((1, D_OUT_K), lambda i: (0, 0)),
        ],
        out_specs=[
            pl.BlockSpec((blk, D_OUT_K), lambda i: (i, 0)),
            pl.BlockSpec((blk, D_OUT_K), lambda i: (i, 0)),
        ],
        out_shape=[
            jax.ShapeDtypeStruct((N_NODES_K, D_OUT_K), jnp.float32),
            jax.ShapeDtypeStruct((N_NODES_K, D_OUT_K), jnp.float32),
        ],
    )(nodes, w_r, w_s, w_g, g, b)


_SC_CHUNK = 40


_SLICES = (192000, 128000)


def _gsum_sc(pre_r, pre_s, receivers, senders, slice_e):
    info = plsc.get_sparse_core_info()
    nc, ns = info.num_cores, info.num_subcores
    nw = nc * ns
    e_per_w = slice_e // nw
    n_chunks = e_per_w // _SC_CHUNK
    n_pairs = n_chunks // 2
    mesh = plsc.VectorSubcoreMesh(core_axis_name="c", subcore_axis_name="s")
    C = _SC_CHUNK

    @functools.partial(
        pl.kernel,
        mesh=mesh,
        out_type=jax.ShapeDtypeStruct((slice_e, D_OUT_K), jnp.float32),
        scratch_types=[
            pltpu.VMEM((e_per_w,), jnp.int32),
            pltpu.VMEM((e_per_w,), jnp.int32),
            [pltpu.VMEM((C, D_OUT_K), jnp.float32) for _ in range(2)],
            [pltpu.VMEM((C, D_OUT_K), jnp.float32) for _ in range(2)],
            [pltpu.VMEM((C, D_OUT_K), jnp.float32) for _ in range(2)],
            [pltpu.SemaphoreType.DMA for _ in range(2)],
            [pltpu.SemaphoreType.DMA for _ in range(2)],
            [pltpu.SemaphoreType.DMA for _ in range(2)],
        ],
    )
    def gsum_kernel(prer_hbm, pres_hbm, recv_hbm, send_hbm, out_hbm,
                    idx_r, idx_s, rows_r, rows_s, outb, sem_r, sem_s,
                    sem_o):
        wid = lax.axis_index("s") * nc + lax.axis_index("c")
        base = wid * e_per_w

        def fire_gathers(c, b):
            pltpu.async_copy(prer_hbm.at[idx_r.at[pl.ds(c * C, C)]],
                             rows_r[b], sem_r[b])
            pltpu.async_copy(pres_hbm.at[idx_s.at[pl.ds(c * C, C)]],
                             rows_s[b], sem_s[b])

        def wait_gathers(c, b):
            pltpu.make_async_copy(prer_hbm.at[idx_r.at[pl.ds(c * C, C)]],
                                  rows_r[b], sem_r[b]).wait()
            pltpu.make_async_copy(pres_hbm.at[idx_s.at[pl.ds(c * C, C)]],
                                  rows_s[b], sem_s[b]).wait()

        def compute(b):
            @plsc.parallel_loop(0, C, step=1, unroll=4)
            def row_body(i):
                for j in range(D_OUT_K // 16):
                    sl = pl.ds(j * 16, 16)
                    outb[b][i, sl] = rows_r[b][i, sl] + rows_s[b][i, sl]

        def fire_out(c, b):
            pltpu.async_copy(outb[b], out_hbm.at[pl.ds(base + c * C, C)],
                             sem_o[b])

        def wait_out(c, b):
            pltpu.make_async_copy(outb[b],
                                  out_hbm.at[pl.ds(base + c * C, C)],
                                  sem_o[b]).wait()

        pltpu.sync_copy(recv_hbm.at[pl.ds(base, e_per_w)], idx_r)
        pltpu.sync_copy(send_hbm.at[pl.ds(base, e_per_w)], idx_s)
        fire_gathers(0, 0)

        def pair_body(g, carry):
            c0 = 2 * g
            c1 = c0 + 1
            fire_gathers(c1, 1)
            wait_gathers(c0, 0)

            @pl.when(g > 0)
            def _():
                wait_out(c0 - 2, 0)

            compute(0)
            fire_out(c0, 0)

            @pl.when(c0 + 2 < n_chunks)
            def _():
                fire_gathers(c0 + 2, 0)

            wait_gathers(c1, 1)

            @pl.when(g > 0)
            def _():
                wait_out(c1 - 2, 1)

            compute(1)
            fire_out(c1, 1)
            return carry

        lax.fori_loop(0, n_pairs, pair_body, 0)
        if n_chunks % 2:
            wait_gathers(n_chunks - 1, 0)
            wait_out(n_chunks - 3, 0)
            compute(0)
            fire_out(n_chunks - 1, 0)
            wait_out(n_chunks - 2, 1)
            wait_out(n_chunks - 1, 0)
        else:
            wait_out(n_chunks - 2, 0)
            wait_out(n_chunks - 1, 1)

    return gsum_kernel(pre_r, pre_s, receivers, senders)


def _combine_body(edges_ref, we_ref, gsum_ref, out_ref):
    ep = jnp.dot(edges_ref[...], we_ref[...],
                 preferred_element_type=jnp.float32)
    out_ref[...] = jnp.maximum(ep + gsum_ref[...], 0.0)


def _combine_first(edges, w_e, gsum0):
    blk = 4000
    grid = _SLICES[0] // blk
    return pl.pallas_call(
        _combine_body,
        grid=(grid,),
        in_specs=[
            pl.BlockSpec((blk, D_EDGE_K), lambda i: (i, 0)),
            pl.BlockSpec((D_EDGE_K, D_OUT_K), lambda i: (0, 0)),
            pl.BlockSpec((blk, D_OUT_K), lambda i: (i, 0)),
        ],
        out_specs=pl.BlockSpec((blk, D_OUT_K), lambda i: (i, 0)),
        out_shape=jax.ShapeDtypeStruct((N_EDGES_K, D_OUT_K), jnp.float32),
    )(edges, w_e, gsum0)


def _combine_slice(edges, w_e, gsum_k, out_prev, row_off, slice_e):
    blk = 4000
    grid = slice_e // blk
    off = row_off // blk

    def _body(edges_ref, we_ref, gsum_ref, _, out_ref):
        _combine_body(edges_ref, we_ref, gsum_ref, out_ref)

    return pl.pallas_call(
        _body,
        grid=(grid,),
        in_specs=[
            pl.BlockSpec((blk, D_EDGE_K), lambda i: (off + i, 0)),
            pl.BlockSpec((D_EDGE_K, D_OUT_K), lambda i: (0, 0)),
            pl.BlockSpec((blk, D_OUT_K), lambda i: (i, 0)),
            pl.BlockSpec(memory_space=pl.ANY),
        ],
        out_specs=pl.BlockSpec((blk, D_OUT_K), lambda i: (off + i, 0)),
        out_shape=jax.ShapeDtypeStruct((N_EDGES_K, D_OUT_K), jnp.float32),
        input_output_aliases={3: 0},
    )(edges, w_e, gsum_k, out_prev)


def kernel(nodes, edges, globals_attr, senders, receivers, W, b):
    w_e = W[:D_EDGE_K]
    w_r = W[D_EDGE_K:D_EDGE_K + D_FEAT_K]
    w_s = W[D_EDGE_K + D_FEAT_K:D_EDGE_K + 2 * D_FEAT_K]
    w_g = W[D_EDGE_K + 2 * D_FEAT_K:]
    b2 = b.reshape(1, D_OUT_K)
    pre_r, pre_s = _make_tables(nodes, w_r, w_s, w_g, globals_attr, b2)
    offs = [0]
    for se in _SLICES[:-1]:
        offs.append(offs[-1] + se)
    gsums = [
        _gsum_sc(pre_r, pre_s,
                 lax.slice(receivers, (o,), (o + se,)),
                 lax.slice(senders, (o,), (o + se,)), se)
        for o, se in zip(offs, _SLICES)
    ]
    out = _combine_first(edges, w_e, gsums[0])
    for k in range(1, len(_SLICES)):
        out = _combine_slice(edges, w_e, gsums[k], out, offs[k], _SLICES[k])
    return out


# speedup vs baseline: 1.0059x; 1.0059x over previous
import functools

import jax
import jax.numpy as jnp
from jax import lax
from jax.experimental import pallas as pl
from jax.experimental.pallas import tpu as pltpu
from jax.experimental.pallas import tpu_sc as plsc

N_NODES_K = 10000
N_EDGES_K = 320000
D_FEAT_K = 128
D_EDGE_K = 16
D_OUT_K = 128


def _tables_body(nodes_ref, wr_ref, ws_ref, wg_ref, g_ref, b_ref,
                 prer_ref, pres_ref):
    gvec = jnp.dot(g_ref[...], wg_ref[...],
                   preferred_element_type=jnp.float32) + b_ref[...]
    n = nodes_ref[...]
    prer_ref[...] = jnp.dot(n, wr_ref[...],
                            preferred_element_type=jnp.float32) + gvec
    pres_ref[...] = jnp.dot(n, ws_ref[...],
                            preferred_element_type=jnp.float32)


def _make_tables(nodes, w_r, w_s, w_g, g, b):
    blk = 2000
    grid = N_NODES_K // blk
    return pl.pallas_call(
        _tables_body,
        grid=(grid,),
        in_specs=[
            pl.BlockSpec((blk, D_FEAT_K), lambda i: (i, 0)),
            pl.BlockSpec((D_FEAT_K, D_OUT_K), lambda i: (0, 0)),
            pl.BlockSpec((D_FEAT_K, D_OUT_K), lambda i: (0, 0)),
            pl.BlockSpec((D_FEAT_K, D_OUT_K), lambda i: (0, 0)),
            pl.BlockSpec((1, D_FEAT_K), lambda i: (0, 0)),
            pl.BlockSpec((1, D_OUT_K), lambda i: (0, 0)),
        ],
        out_specs=[
            pl.BlockSpec((blk, D_OUT_K), lambda i: (i, 0)),
            pl.BlockSpec((blk, D_OUT_K), lambda i: (i, 0)),
        ],
        out_shape=[
            jax.ShapeDtypeStruct((N_NODES_K, D_OUT_K), jnp.float32),
            jax.ShapeDtypeStruct((N_NODES_K, D_OUT_K), jnp.float32),
        ],
    )(nodes, w_r, w_s, w_g, g, b)


_SC_CHUNK = 40


_SLICES = (160000, 160000)


def _gsum_sc(pre_r, pre_s, receivers, senders, slice_e):
    info = plsc.get_sparse_core_info()
    nc, ns = info.num_cores, info.num_subcores
    nw = nc * ns
    e_per_w = slice_e // nw
    n_chunks = e_per_w // _SC_CHUNK
    n_pairs = n_chunks // 2
    mesh = plsc.VectorSubcoreMesh(core_axis_name="c", subcore_axis_name="s")
    C = _SC_CHUNK

    @functools.partial(
        pl.kernel,
        mesh=mesh,
        out_type=jax.ShapeDtypeStruct((slice_e, D_OUT_K), jnp.float32),
        scratch_types=[
            pltpu.VMEM((e_per_w,), jnp.int32),
            pltpu.VMEM((e_per_w,), jnp.int32),
            [pltpu.VMEM((C, D_OUT_K), jnp.float32) for _ in range(2)],
            [pltpu.VMEM((C, D_OUT_K), jnp.float32) for _ in range(2)],
            [pltpu.VMEM((C, D_OUT_K), jnp.float32) for _ in range(2)],
            [pltpu.SemaphoreType.DMA for _ in range(2)],
            [pltpu.SemaphoreType.DMA for _ in range(2)],
            [pltpu.SemaphoreType.DMA for _ in range(2)],
        ],
    )
    def gsum_kernel(prer_hbm, pres_hbm, recv_hbm, send_hbm, out_hbm,
                    idx_r, idx_s, rows_r, rows_s, outb, sem_r, sem_s,
                    sem_o):
        wid = lax.axis_index("s") * nc + lax.axis_index("c")
        base = wid * e_per_w

        def fire_gathers(c, b):
            pltpu.async_copy(prer_hbm.at[idx_r.at[pl.ds(c * C, C)]],
                             rows_r[b], sem_r[b])
            pltpu.async_copy(pres_hbm.at[idx_s.at[pl.ds(c * C, C)]],
                             rows_s[b], sem_s[b])

        def wait_gathers(c, b):
            pltpu.make_async_copy(prer_hbm.at[idx_r.at[pl.ds(c * C, C)]],
                                  rows_r[b], sem_r[b]).wait()
            pltpu.make_async_copy(pres_hbm.at[idx_s.at[pl.ds(c * C, C)]],
                                  rows_s[b], sem_s[b]).wait()

        def compute(b):
            @plsc.parallel_loop(0, C, step=1, unroll=4)
            def row_body(i):
                for j in range(D_OUT_K // 16):
                    sl = pl.ds(j * 16, 16)
                    outb[b][i, sl] = rows_r[b][i, sl] + rows_s[b][i, sl]

        def fire_out(c, b):
            pltpu.async_copy(outb[b], out_hbm.at[pl.ds(base + c * C, C)],
                             sem_o[b])

        def wait_out(c, b):
            pltpu.make_async_copy(outb[b],
                                  out_hbm.at[pl.ds(base + c * C, C)],
                                  sem_o[b]).wait()

        pltpu.sync_copy(recv_hbm.at[pl.ds(base, e_per_w)], idx_r)
        pltpu.sync_copy(send_hbm.at[pl.ds(base, e_per_w)], idx_s)
        fire_gathers(0, 0)

        def pair_body(g, carry):
            c0 = 2 * g
            c1 = c0 + 1
            fire_gathers(c1, 1)
            wait_gathers(c0, 0)

            @pl.when(g > 0)
            def _():
                wait_out(c0 - 2, 0)

            compute(0)
            fire_out(c0, 0)

            @pl.when(c0 + 2 < n_chunks)
            def _():
                fire_gathers(c0 + 2, 0)

            wait_gathers(c1, 1)

            @pl.when(g > 0)
            def _():
                wait_out(c1 - 2, 1)

            compute(1)
            fire_out(c1, 1)
            return carry

        lax.fori_loop(0, n_pairs, pair_body, 0)
        if n_chunks % 2:
            wait_gathers(n_chunks - 1, 0)
            wait_out(n_chunks - 3, 0)
            compute(0)
            fire_out(n_chunks - 1, 0)
            wait_out(n_chunks - 2, 1)
            wait_out(n_chunks - 1, 0)
        else:
            wait_out(n_chunks - 2, 0)
            wait_out(n_chunks - 1, 1)

    return gsum_kernel(pre_r, pre_s, receivers, senders)


def _combine_body(edges_ref, we_ref, gsum_ref, out_ref):
    ep = jnp.dot(edges_ref[...], we_ref[...],
                 preferred_element_type=jnp.float32)
    out_ref[...] = jnp.maximum(ep + gsum_ref[...], 0.0)


def _combine_first(edges, w_e, gsum0):
    blk = 4000
    grid = _SLICES[0] // blk
    return pl.pallas_call(
        _combine_body,
        grid=(grid,),
        in_specs=[
            pl.BlockSpec((blk, D_EDGE_K), lambda i: (i, 0)),
            pl.BlockSpec((D_EDGE_K, D_OUT_K), lambda i: (0, 0)),
            pl.BlockSpec((blk, D_OUT_K), lambda i: (i, 0)),
        ],
        out_specs=pl.BlockSpec((blk, D_OUT_K), lambda i: (i, 0)),
        out_shape=jax.ShapeDtypeStruct((N_EDGES_K, D_OUT_K), jnp.float32),
    )(edges, w_e, gsum0)


def _combine_slice(edges, w_e, gsum_k, out_prev, row_off, slice_e):
    blk = 4000
    grid = slice_e // blk
    off = row_off // blk

    def _body(edges_ref, we_ref, gsum_ref, _, out_ref):
        _combine_body(edges_ref, we_ref, gsum_ref, out_ref)

    return pl.pallas_call(
        _body,
        grid=(grid,),
        in_specs=[
            pl.BlockSpec((blk, D_EDGE_K), lambda i: (off + i, 0)),
            pl.BlockSpec((D_EDGE_K, D_OUT_K), lambda i: (0, 0)),
            pl.BlockSpec((blk, D_OUT_K), lambda i: (i, 0)),
            pl.BlockSpec(memory_space=pl.ANY),
        ],
        out_specs=pl.BlockSpec((blk, D_OUT_K), lambda i: (off + i, 0)),
        out_shape=jax.ShapeDtypeStruct((N_EDGES_K, D_OUT_K), jnp.float32),
        input_output_aliases={3: 0},
    )(edges, w_e, gsum_k, out_prev)


def kernel(nodes, edges, globals_attr, senders, receivers, W, b):
    w_e = W[:D_EDGE_K]
    w_r = W[D_EDGE_K:D_EDGE_K + D_FEAT_K]
    w_s = W[D_EDGE_K + D_FEAT_K:D_EDGE_K + 2 * D_FEAT_K]
    w_g = W[D_EDGE_K + 2 * D_FEAT_K:]
    b2 = b.reshape(1, D_OUT_K)
    pre_r, pre_s = _make_tables(nodes, w_r, w_s, w_g, globals_attr, b2)
    offs = [0]
    for se in _SLICES[:-1]:
        offs.append(offs[-1] + se)
    gsums = [
        _gsum_sc(pre_r, pre_s,
                 lax.slice(receivers, (o,), (o + se,)),
                 lax.slice(senders, (o,), (o + se,)), se)
        for o, se in zip(offs, _SLICES)
    ]
    out = _combine_first(edges, w_e, gsums[0])
    for k in range(1, len(_SLICES)):
        out = _combine_slice(edges, w_e, gsums[k], out, offs[k], _SLICES[k])
    return out
